# baseline (speedup 1.0000x reference)
import functools

import jax
import jax.numpy as jnp
from jax import lax
from jax.experimental import pallas as pl
from jax.experimental.pallas import tpu as pltpu
from jax.experimental.pallas import tpu_sc as plsc

BP = 128
TE = 8000
TN = 2000
NPAD = 10240
NW = 32
CH = 2000


def _elu(x):
    return jnp.where(x > 0, x, jnp.exp(jnp.minimum(x, 0.0)) - 1.0)


def _edge_body(eb_ref, ea_ref, instr_ref, we_ref, wr_ref, v_ref):
    eb = eb_ref[0]
    onehot_t = (lax.broadcasted_iota(jnp.int32, (BP, TE), 0) == eb)
    onehot_t = onehot_t.astype(jnp.float32)
    instr_g = lax.dot_general(
        onehot_t, instr_ref[...], (((0,), (0,)), ((), ())),
        preferred_element_type=jnp.float32,
        precision=lax.Precision.HIGHEST)
    y = jnp.dot(instr_g * ea_ref[...], we_ref[...],
                preferred_element_type=jnp.float32,
                precision=lax.Precision.DEFAULT)
    v = lax.dot_general(
        wr_ref[...], _elu(y), (((1,), (1,)), ((), ())),
        preferred_element_type=jnp.float32,
        precision=lax.Precision.HIGHEST)
    v_ref[...] = v.reshape(1, 1, TE)


def _edge_stage(eb3, edge_attrs, instr_pad, weight_edge, wr2,
                interpret=False):
    E, H = edge_attrs.shape
    grid = E // TE
    return pl.pallas_call(
        _edge_body,
        grid=(grid,),
        in_specs=[
            pl.BlockSpec((1, 1, TE), lambda i: (i, 0, 0)),
            pl.BlockSpec((TE, H), lambda i: (i, 0)),
            pl.BlockSpec((BP, H), lambda i: (0, 0)),
            pl.BlockSpec((H, H), lambda i: (0, 0)),
            pl.BlockSpec((1, H), lambda i: (0, 0)),
        ],
        out_specs=pl.BlockSpec((1, 1, TE), lambda i: (i, 0, 0)),
        out_shape=jax.ShapeDtypeStruct((grid, 1, TE), jnp.float32),
        interpret=interpret,
    )(eb3, edge_attrs, instr_pad, weight_edge, wr2)


def _node_body(ni_ref, na_ref, gt_ref, wp_ref, wn_ref, ls_ref):
    P = na_ref.shape[1]
    ni = ni_ref[0]
    onehot_t = (lax.broadcasted_iota(jnp.int32, (BP, TN), 0) == ni)
    onehot_t = onehot_t.astype(jnp.float32)
    g = lax.dot_general(
        onehot_t, gt_ref[...], (((0,), (0,)), ((), ())),
        preferred_element_type=jnp.float32,
        precision=lax.Precision.HIGHEST)
    H = gt_ref.shape[1] - BP
    instr_g = g[:, :H]
    prop_g = g[:, H:]
    acc = None
    for p in range(P):
        t = jnp.dot((prop_g[:, p:p + 1] * instr_g) * na_ref[:, p, :],
                    wp_ref[p], preferred_element_type=jnp.float32,
                    precision=lax.Precision.DEFAULT)
        acc = t if acc is None else acc + t
    ls = lax.dot_general(
        wn_ref[...], _elu(acc), (((1,), (1,)), ((), ())),
        preferred_element_type=jnp.float32,
        precision=lax.Precision.DEFAULT)
    ls_ref[...] = ls.reshape(1, 1, TN)


def _node_stage(ni3, node_attrs, gt, wp, wn2, interpret=False):
    N, P, H = node_attrs.shape
    GT = gt.shape[1]
    grid = N // TN
    return pl.pallas_call(
        _node_body,
        grid=(grid,),
        in_specs=[
            pl.BlockSpec((1, 1, TN), lambda i: (i, 0, 0)),
            pl.BlockSpec((TN, P, H), lambda i: (i, 0, 0)),
            pl.BlockSpec((BP, GT), lambda i: (0, 0)),
            pl.BlockSpec((P, H, H), lambda i: (0, 0, 0)),
            pl.BlockSpec((1, H), lambda i: (0, 0)),
        ],
        out_specs=pl.BlockSpec((1, 1, TN), lambda i: (i, 0, 0)),
        out_shape=jax.ShapeDtypeStruct((grid, 1, TN), jnp.float32),
        interpret=interpret,
    )(ni3, node_attrs, gt, wp, wn2)


def _sc_body(v_hbm, src_hbm, dst_hbm, dist_hbm, out_hbm,
             dist_v, acc_v, vv, sv, dv):
    E = v_hbm.shape[0]
    ew = E // NW
    nc = 2
    wid = lax.axis_index("s") * nc + lax.axis_index("c")
    pltpu.sync_copy(dist_hbm, dist_v)

    def zbody(i, _):
        acc_v[pl.ds(i * 16, 16)] = jnp.zeros((16,), jnp.float32)
        return 0
    lax.fori_loop(0, NPAD // 16, zbody, 0)

    def cbody(c, _):
        base = wid * ew + c * CH
        pltpu.sync_copy(v_hbm.at[pl.ds(base, CH)], vv)
        pltpu.sync_copy(src_hbm.at[pl.ds(base, CH)], sv)
        pltpu.sync_copy(dst_hbm.at[pl.ds(base, CH)], dv)

        def lbody(j, _):
            s = sv[pl.ds(j * 16, 16)]
            d = dv[pl.ds(j * 16, 16)]
            val = vv[pl.ds(j * 16, 16)]
            g = plsc.load_gather(dist_v, [s])
            plsc.addupdate_scatter(acc_v, [d], g * val)
            return 0
        lax.fori_loop(0, CH // 16, lbody, 0)
        return 0
    lax.fori_loop(0, ew // CH, cbody, 0)
    pltpu.sync_copy(acc_v, out_hbm.at[wid])


def _sc_scatter_stage(v, src, dst, dist):
    N = dist.shape[0]
    mesh = plsc.VectorSubcoreMesh(core_axis_name="c", subcore_axis_name="s")
    f = functools.partial(
        pl.kernel,
        out_type=jax.ShapeDtypeStruct((NW, NPAD), jnp.float32),
        mesh=mesh,
        compiler_params=pltpu.CompilerParams(needs_layout_passes=False),
        scratch_types=[
            pltpu.VMEM((N,), jnp.float32),
            pltpu.VMEM((NPAD,), jnp.float32),
            pltpu.VMEM((CH,), jnp.float32),
            pltpu.VMEM((CH,), jnp.int32),
            pltpu.VMEM((CH,), jnp.int32),
        ],
    )(_sc_body)
    return f(v, src, dst, dist)


def _combine_body(seg_ref, ls_ref, lrp_ref, rs_ref, out_ref):
    seg = seg_ref[...]
    mask = lax.broadcasted_iota(jnp.int32, (BP, NPAD), 0) == seg
    ls = jnp.broadcast_to(ls_ref[...], (BP, NPAD))
    lr = jnp.broadcast_to(jnp.sum(lrp_ref[...], axis=0, keepdims=True),
                          (BP, NPAD))

    def seg_softmax(l):
        m = jnp.max(jnp.where(mask, l, -jnp.inf), axis=1, keepdims=True)
        m = jnp.where(m == -jnp.inf, 0.0, m)
        e = jnp.where(mask, jnp.exp(l - m), 0.0)
        s = jnp.sum(e, axis=1, keepdims=True)
        return e / jnp.where(s == 0.0, 1.0, s)

    p_st = seg_softmax(ls)
    p_rel = seg_softmax(lr)
    rs = rs_ref[...]
    out_ref[...] = jnp.sum(rs * p_rel + (1.0 - rs) * p_st,
                           axis=0, keepdims=True)


def _combine_stage(seg_pad, ls_pad, lrp, rs_pad, interpret=False):
    return pl.pallas_call(
        _combine_body,
        out_shape=jax.ShapeDtypeStruct((1, NPAD), jnp.float32),
        interpret=interpret,
    )(seg_pad, ls_pad, lrp, rs_pad)


def kernel(node_attrs, edge_attrs, instruction_batch, distribution,
           node_prop_similarities, relation_similarity,
           weight_node_properties, weight_edge, weight_node_score,
           weight_relation_score, edge_indices, node_indices,
           edge_batch_indices):
    N, P, H = node_attrs.shape
    E = edge_attrs.shape[0]
    B = instruction_batch.shape[0]

    instr_pad = jnp.pad(instruction_batch, ((0, BP - B), (0, 0)))
    nps_pad = jnp.pad(node_prop_similarities, ((0, BP - B), (0, BP - P)))
    rs_pad = jnp.pad(relation_similarity, (0, BP - B)).reshape(BP, 1)
    wr2 = weight_relation_score.reshape(1, H)
    wn2 = weight_node_score.reshape(1, H)
    eb3 = edge_batch_indices.reshape(E // TE, 1, TE)
    ni3 = node_indices.reshape(N // TN, 1, TN)

    gt = jnp.concatenate([instr_pad, nps_pad], axis=1)

    v = _edge_stage(eb3, edge_attrs, instr_pad, weight_edge, wr2)
    ls = _node_stage(ni3, node_attrs, gt,
                     weight_node_properties, wn2)
    lrp = _sc_scatter_stage(v.reshape(E), edge_indices[0], edge_indices[1],
                            distribution)

    seg_pad = jnp.pad(node_indices, (0, NPAD - N),
                      constant_values=BP - 1).reshape(1, NPAD)
    ls_pad = jnp.pad(ls.reshape(N), (0, NPAD - N)).reshape(1, NPAD)
    out = _combine_stage(seg_pad, ls_pad, lrp, rs_pad)
    return out[0, :N]


# speedup vs baseline: 1.3832x; 1.3832x over previous
import functools

import jax
import jax.numpy as jnp
from jax import lax
from jax.experimental import pallas as pl
from jax.experimental.pallas import tpu as pltpu
from jax.experimental.pallas import tpu_sc as plsc

BP = 128
TE = 8000
TN = 2000
NPAD = 10240
NW = 32
CH = 2000


def _elu(x):
    return jnp.where(x > 0, x, jnp.exp(jnp.minimum(x, 0.0)) - 1.0)


def _edge_body(eb_ref, ea_ref, instr_ref, we_ref, wr_ref, v_ref):
    eb = eb_ref[...]
    onehot = (lax.broadcasted_iota(jnp.int32, (TE, BP), 1) == eb)
    onehot = onehot.astype(jnp.float32)
    instr_g = jnp.dot(onehot, instr_ref[...],
                      preferred_element_type=jnp.float32,
                      precision=lax.Precision.HIGHEST)
    y = jnp.dot(instr_g * ea_ref[...], we_ref[...],
                preferred_element_type=jnp.float32,
                precision=lax.Precision.DEFAULT)
    v_ref[...] = jnp.sum(_elu(y) * wr_ref[...], axis=1, keepdims=True)


def _edge_stage(eb2, edge_attrs, instr_pad, weight_edge, wr2,
                interpret=False):
    E, H = edge_attrs.shape
    grid = E // TE
    return pl.pallas_call(
        _edge_body,
        grid=(grid,),
        in_specs=[
            pl.BlockSpec((TE, 1), lambda i: (i, 0)),
            pl.BlockSpec((TE, H), lambda i: (i, 0)),
            pl.BlockSpec((BP, H), lambda i: (0, 0)),
            pl.BlockSpec((H, H), lambda i: (0, 0)),
            pl.BlockSpec((1, H), lambda i: (0, 0)),
        ],
        out_specs=pl.BlockSpec((TE, 1), lambda i: (i, 0)),
        out_shape=jax.ShapeDtypeStruct((E, 1), jnp.float32),
        interpret=interpret,
    )(eb2, edge_attrs, instr_pad, weight_edge, wr2)


def _node_body(ni_ref, na_ref, gt_ref, wp_ref, wn_ref, ls_ref):
    P = na_ref.shape[1]
    ni = ni_ref[...]
    onehot = (lax.broadcasted_iota(jnp.int32, (TN, BP), 1) == ni)
    onehot = onehot.astype(jnp.float32)
    g = jnp.dot(onehot, gt_ref[...],
                preferred_element_type=jnp.float32,
                precision=lax.Precision.HIGHEST)
    H = gt_ref.shape[1] - BP
    instr_g = g[:, :H]
    prop_g = g[:, H:]
    acc = None
    for p in range(P):
        t = jnp.dot((prop_g[:, p:p + 1] * instr_g) * na_ref[:, p, :],
                    wp_ref[p], preferred_element_type=jnp.float32,
                    precision=lax.Precision.DEFAULT)
        acc = t if acc is None else acc + t
    ls_ref[...] = jnp.dot(_elu(acc), wn_ref[...],
                          preferred_element_type=jnp.float32,
                          precision=lax.Precision.DEFAULT)


def _node_stage(ni2, node_attrs, gt, wp, wn2, interpret=False):
    N, P, H = node_attrs.shape
    GT = gt.shape[1]
    grid = N // TN
    return pl.pallas_call(
        _node_body,
        grid=(grid,),
        in_specs=[
            pl.BlockSpec((TN, 1), lambda i: (i, 0)),
            pl.BlockSpec((TN, P, H), lambda i: (i, 0, 0)),
            pl.BlockSpec((BP, GT), lambda i: (0, 0)),
            pl.BlockSpec((P, H, H), lambda i: (0, 0, 0)),
            pl.BlockSpec((H, 1), lambda i: (0, 0)),
        ],
        out_specs=pl.BlockSpec((TN, 1), lambda i: (i, 0)),
        out_shape=jax.ShapeDtypeStruct((N, 1), jnp.float32),
        interpret=interpret,
    )(ni2, node_attrs, gt, wp, wn2)


def _sc_body(v_hbm, src_hbm, dst_hbm, dist_hbm, out_hbm,
             dist_v, acc_v, vv, sv, dv):
    E = v_hbm.shape[0]
    ew = E // NW
    nc = 2
    wid = lax.axis_index("s") * nc + lax.axis_index("c")
    pltpu.sync_copy(dist_hbm, dist_v)

    def zbody(i, _):
        acc_v[pl.ds(i * 16, 16)] = jnp.zeros((16,), jnp.float32)
        return 0
    lax.fori_loop(0, NPAD // 16, zbody, 0)

    def cbody(c, _):
        base = wid * ew + c * CH
        pltpu.sync_copy(v_hbm.at[pl.ds(base, CH)], vv)
        pltpu.sync_copy(src_hbm.at[pl.ds(base, CH)], sv)
        pltpu.sync_copy(dst_hbm.at[pl.ds(base, CH)], dv)

        def lbody(j, _):
            s = sv[pl.ds(j * 16, 16)]
            d = dv[pl.ds(j * 16, 16)]
            val = vv[pl.ds(j * 16, 16)]
            g = plsc.load_gather(dist_v, [s])
            plsc.addupdate_scatter(acc_v, [d], g * val)
            return 0
        lax.fori_loop(0, CH // 16, lbody, 0)
        return 0
    lax.fori_loop(0, ew // CH, cbody, 0)
    pltpu.sync_copy(acc_v, out_hbm.at[wid])


def _sc_scatter_stage(v, src, dst, dist):
    N = dist.shape[0]
    mesh = plsc.VectorSubcoreMesh(core_axis_name="c", subcore_axis_name="s")
    f = functools.partial(
        pl.kernel,
        out_type=jax.ShapeDtypeStruct((NW, NPAD), jnp.float32),
        mesh=mesh,
        compiler_params=pltpu.CompilerParams(needs_layout_passes=False),
        scratch_types=[
            pltpu.VMEM((N,), jnp.float32),
            pltpu.VMEM((NPAD,), jnp.float32),
            pltpu.VMEM((CH,), jnp.float32),
            pltpu.VMEM((CH,), jnp.int32),
            pltpu.VMEM((CH,), jnp.int32),
        ],
    )(_sc_body)
    return f(v, src, dst, dist)


def _combine_body(seg_ref, ls_ref, lrp_ref, rs_ref, out_ref):
    seg = seg_ref[...]
    mask = lax.broadcasted_iota(jnp.int32, (BP, NPAD), 0) == seg
    ls = jnp.broadcast_to(ls_ref[...], (BP, NPAD))
    lr = jnp.broadcast_to(jnp.sum(lrp_ref[...], axis=0, keepdims=True),
                          (BP, NPAD))

    def seg_softmax(l):
        m = jnp.max(jnp.where(mask, l, -jnp.inf), axis=1, keepdims=True)
        m = jnp.where(m == -jnp.inf, 0.0, m)
        e = jnp.where(mask, jnp.exp(l - m), 0.0)
        s = jnp.sum(e, axis=1, keepdims=True)
        return e / jnp.where(s == 0.0, 1.0, s)

    p_st = seg_softmax(ls)
    p_rel = seg_softmax(lr)
    rs = rs_ref[...]
    out_ref[...] = jnp.sum(rs * p_rel + (1.0 - rs) * p_st,
                           axis=0, keepdims=True)


def _combine_stage(seg_pad, ls_pad, lrp, rs_pad, interpret=False):
    return pl.pallas_call(
        _combine_body,
        out_shape=jax.ShapeDtypeStruct((1, NPAD), jnp.float32),
        interpret=interpret,
    )(seg_pad, ls_pad, lrp, rs_pad)


def kernel(node_attrs, edge_attrs, instruction_batch, distribution,
           node_prop_similarities, relation_similarity,
           weight_node_properties, weight_edge, weight_node_score,
           weight_relation_score, edge_indices, node_indices,
           edge_batch_indices):
    N, P, H = node_attrs.shape
    E = edge_attrs.shape[0]
    B = instruction_batch.shape[0]

    instr_pad = jnp.pad(instruction_batch, ((0, BP - B), (0, 0)))
    nps_pad = jnp.pad(node_prop_similarities, ((0, BP - B), (0, BP - P)))
    rs_pad = jnp.pad(relation_similarity, (0, BP - B)).reshape(BP, 1)
    wr2 = weight_relation_score.reshape(1, H)
    wn2 = weight_node_score.reshape(H, 1)
    eb2 = edge_batch_indices.reshape(E, 1)
    ni2 = node_indices.reshape(N, 1)

    gt = jnp.concatenate([instr_pad, nps_pad], axis=1)

    v = _edge_stage(eb2, edge_attrs, instr_pad, weight_edge, wr2)
    ls = _node_stage(ni2, node_attrs, gt,
                     weight_node_properties, wn2)
    lrp = _sc_scatter_stage(v.reshape(E), edge_indices[0], edge_indices[1],
                            distribution)

    seg_pad = jnp.pad(node_indices, (0, NPAD - N),
                      constant_values=BP - 1).reshape(1, NPAD)
    ls_pad = jnp.pad(ls.reshape(N), (0, NPAD - N)).reshape(1, NPAD)
    out = _combine_stage(seg_pad, ls_pad, lrp, rs_pad)
    return out[0, :N]
